# baseline (speedup 1.0000x reference)
import functools

import jax
import jax.numpy as jnp
from jax import lax
from jax.experimental import pallas as pl
from jax.experimental.pallas import tpu as pltpu
from jax.experimental.pallas import tpu_sc as plsc

TOKENS = 16384
EMBED = 2048
NUM_EXPERTS = 64
ACTIVE_EXPERTS = 8

BLOCK_T = 2048

_NC, _NS, _L = 2, 16, 16
_NW = _NC * _NS
_C = TOKENS // _NW


def _scores_kernel(x_ref, w_ref, b_ref, s_ref):
    s = jax.lax.dot_general(
        x_ref[...], w_ref[...], (((1,), (1,)), ((), ())),
        preferred_element_type=jnp.float32,
    )
    s_ref[...] = s + b_ref[...]


def _merge16(ka, ia, kb, ib):
    kbr = lax.rev(kb, (0,))
    ibr = lax.rev(ib, (0,))
    sel = (ka > kbr) | ((ka == kbr) & (ia < ibr))
    mk = jnp.where(sel, ka, kbr)
    mi = jnp.where(sel, ia, ibr)
    return plsc.sort_key_val(mk, mi, descending=True)


def _sc_router_impl(s_hbm, out_hbm, idx_hbm, sin, sout, sidx):
    wid = lax.axis_index("s") * _NC + lax.axis_index("c")
    base = wid * _C
    pltpu.sync_copy(s_hbm.at[pl.ds(base, _C)], sin)

    iot = lax.iota(jnp.int32, 16)
    lane8 = iot < 8

    def body(t, carry):
        k0, i0 = plsc.sort_key_val(sin[t, 0], iot, descending=True)
        k1, i1 = plsc.sort_key_val(sin[t, 1], iot + 16, descending=True)
        k2, i2 = plsc.sort_key_val(sin[t, 2], iot + 32, descending=True)
        k3, i3 = plsc.sort_key_val(sin[t, 3], iot + 48, descending=True)
        ka, ia = _merge16(k0, i0, k1, i1)
        kb, ib = _merge16(k2, i2, k3, i3)
        kt, it = _merge16(ka, ia, kb, ib)

        ev = jnp.where(lane8, jnp.exp(kt), 0.0)
        denom = jnp.sum(ev) + jnp.float32(NUM_EXPERTS - ACTIVE_EXPERTS)
        bb = jnp.full((16,), 1.0, jnp.float32) / denom
        sout[t, 0] = bb
        sout[t, 1] = bb
        sout[t, 2] = bb
        sout[t, 3] = bb
        tvec = jnp.full((16,), 0, jnp.int32) + t
        plsc.store_scatter(
            sout,
            [tvec, lax.shift_right_logical(it, 4), it & 15],
            ev / denom,
            mask=lane8,
        )
        sidx[t] = it
        return carry

    lax.fori_loop(0, _C, body, 0)

    pltpu.sync_copy(sout, out_hbm.at[pl.ds(base, _C)])
    pltpu.sync_copy(sidx, idx_hbm.at[pl.ds(base, _C)])


@functools.lru_cache(maxsize=1)
def _get_sc_router():
    return pl.kernel(
        _sc_router_impl,
        mesh=plsc.VectorSubcoreMesh(core_axis_name="c", subcore_axis_name="s"),
        out_type=[
            jax.ShapeDtypeStruct((TOKENS, 4, 16), jnp.float32),
            jax.ShapeDtypeStruct((TOKENS, 16), jnp.int32),
        ],
        scratch_types=[
            pltpu.VMEM((_C, 4, 16), jnp.float32),
            pltpu.VMEM((_C, 4, 16), jnp.float32),
            pltpu.VMEM((_C, 16), jnp.int32),
        ],
        compiler_params=pltpu.CompilerParams(needs_layout_passes=False, use_tc_tiling_on_sc=False),
    )


@jax.jit
def kernel(inputs, W, b):
    b2 = b.reshape(1, NUM_EXPERTS)
    grid = (TOKENS // BLOCK_T,)
    scores = pl.pallas_call(
        _scores_kernel,
        grid=grid,
        in_specs=[
            pl.BlockSpec((BLOCK_T, EMBED), lambda i: (i, 0)),
            pl.BlockSpec((NUM_EXPERTS, EMBED), lambda i: (0, 0)),
            pl.BlockSpec((1, NUM_EXPERTS), lambda i: (0, 0)),
        ],
        out_specs=pl.BlockSpec((BLOCK_T, NUM_EXPERTS), lambda i: (i, 0)),
        out_shape=jax.ShapeDtypeStruct((TOKENS, NUM_EXPERTS), jnp.float32),
    )(inputs, W, b2)
    out3, idx16 = _get_sc_router()(scores.reshape(TOKENS, 4, 16))
    return (out3.reshape(TOKENS, NUM_EXPERTS), idx16[:, :ACTIVE_EXPERTS])
